# baseline (speedup 1.0000x reference)
import jax
import jax.numpy as jnp
from jax.experimental import pallas as pl
from jax.experimental.pallas import tpu as pltpu


def _scale_kernel(kr_ref, x_ref, out_ref):
    out_ref[...] = x_ref[...] * kr_ref[0]


def _fill_kernel(kr_ref, mask_ref):
    mask_ref[...] = jnp.full(mask_ref.shape, kr_ref[0], dtype=mask_ref.dtype)


def _pick_tile(rows, target):
    t = target
    while t > 8 and rows % t != 0:
        t //= 2
    return t if rows % t == 0 else rows


def kernel(x, W1, b1, W2, b2, keep_ratio):
    Bs, Ns, Cs = x.shape
    rows = Bs * Ns
    xf = x.reshape(rows, Cs)
    kr = jnp.asarray(keep_ratio, x.dtype).reshape(1)

    t_out = _pick_tile(rows, 4608)
    out = pl.pallas_call(
        _scale_kernel,
        grid=(rows // t_out,),
        in_specs=[
            pl.BlockSpec(memory_space=pltpu.SMEM),
            pl.BlockSpec((t_out, Cs), lambda i: (i, 0)),
        ],
        out_specs=pl.BlockSpec((t_out, Cs), lambda i: (i, 0)),
        out_shape=jax.ShapeDtypeStruct((rows, Cs), x.dtype),
        compiler_params=pltpu.CompilerParams(
            dimension_semantics=("parallel",),
        ),
    )(kr, xf)

    t_m = _pick_tile(rows, 9216)
    mask = pl.pallas_call(
        _fill_kernel,
        grid=(rows // t_m,),
        in_specs=[pl.BlockSpec(memory_space=pltpu.SMEM)],
        out_specs=pl.BlockSpec((t_m, Cs), lambda i: (i, 0)),
        out_shape=jax.ShapeDtypeStruct((rows, Cs), x.dtype),
        compiler_params=pltpu.CompilerParams(
            dimension_semantics=("parallel",),
        ),
    )(kr)

    return out.reshape(Bs, Ns, Cs), mask.reshape(Bs, Ns, Cs)
